# Initial kernel scaffold; baseline (speedup 1.0000x reference)
import jax
import jax.numpy as jnp
from jax.experimental import pallas as pl


def kernel(context_tokens, age_table, gender_table):
    raise NotImplementedError("write your pallas kernel here")


# speedup vs baseline: 2.7019x; 2.7019x over previous
import functools

import jax
import jax.numpy as jnp
from jax import lax
from jax.experimental import pallas as pl
from jax.experimental.pallas import tpu as pltpu
from jax.experimental.pallas import tpu_sc as plsc

B = 16384
GD = 2
AD = 4
D = GD + AD
VOCAB = 1000
L = 16

_info = plsc.get_sparse_core_info()
NC, NS = _info.num_cores, _info.num_subcores
NW = NC * NS
BPW = B // NW
NGROUP = BPW // L


def _body(tok_hbm, gtab_hbm, atab_hbm, out_hbm, tok_v, gtab_v, atab_v, out_v):
    wid = lax.axis_index("s") * NC + lax.axis_index("c")
    base = wid * BPW
    pltpu.sync_copy(tok_hbm.at[pl.ds(base * 2, BPW * 2)], tok_v)
    pltpu.sync_copy(gtab_hbm, gtab_v)
    pltpu.sync_copy(atab_hbm, atab_v)

    lanes = lax.iota(jnp.int32, L)
    for g in range(NGROUP):
        rows = lanes + (g * L)
        rows2 = rows * 2
        rows6 = rows * D
        gidx2 = plsc.load_gather(tok_v, [rows2]) * GD
        aidx4 = plsc.load_gather(tok_v, [rows2 + 1]) * AD
        for c in range(GD):
            v = plsc.load_gather(gtab_v, [gidx2 + c])
            plsc.store_scatter(out_v, [rows6 + c], v)
        for c in range(AD):
            v = plsc.load_gather(atab_v, [aidx4 + c])
            plsc.store_scatter(out_v, [rows6 + (GD + c)], v)

    pltpu.sync_copy(out_v, out_hbm.at[pl.ds(base * D, BPW * D)])


_ctx_embed = functools.partial(
    pl.kernel,
    mesh=plsc.VectorSubcoreMesh(core_axis_name="c", subcore_axis_name="s"),
    out_type=jax.ShapeDtypeStruct((B * D,), jnp.float32),
    scratch_types=[
        pltpu.VMEM((BPW * 2,), jnp.int32),
        pltpu.VMEM((VOCAB * GD,), jnp.float32),
        pltpu.VMEM((VOCAB * AD,), jnp.float32),
        pltpu.VMEM((BPW * D,), jnp.float32),
    ],
    compiler_params=pltpu.CompilerParams(needs_layout_passes=False),
)(_body)


@jax.jit
def kernel(context_tokens, age_table, gender_table):
    tok = context_tokens.astype(jnp.int32).reshape(-1)
    flat = _ctx_embed(tok, gender_table.reshape(-1), age_table.reshape(-1))
    return flat.reshape(B, D)


# speedup vs baseline: 5.9787x; 2.2128x over previous
import functools

import jax
import jax.numpy as jnp
from jax import lax
from jax.experimental import pallas as pl
from jax.experimental.pallas import tpu as pltpu
from jax.experimental.pallas import tpu_sc as plsc

B = 16384
GD = 2
AD = 4
D = GD + AD
DP = 8
VOCAB = 1000
VP = 1024
TL = 128
L = 16

_info = plsc.get_sparse_core_info()
NC, NS = _info.num_cores, _info.num_subcores
NW = NC * NS
BPW = B // NW
TPW = BPW // TL
NGROUP = BPW // L

TOK_W = 2 * TL
OUT_W = DP * TL
G_W = GD * TL
A_W = AD * TL


def _body(tok_hbm, gtab_hbm, atab_hbm, out_hbm, tok_v, gtab_v, atab_v, out_v):
    wid = lax.axis_index("s") * NC + lax.axis_index("c")
    pltpu.sync_copy(tok_hbm.at[pl.ds(wid * (TPW * TOK_W), TPW * TOK_W)], tok_v)
    pltpu.sync_copy(gtab_hbm, gtab_v)
    pltpu.sync_copy(atab_hbm, atab_v)

    for g in range(NGROUP):
        t, r = divmod(g * L, TL)
        gidx = tok_v[pl.ds(t * TOK_W + r, L)]
        aidx = tok_v[pl.ds(t * TOK_W + TL + r, L)]
        gbase = (gidx >> 7) * G_W + (gidx & (TL - 1))
        abase = (aidx >> 7) * A_W + (aidx & (TL - 1))
        for c in range(GD):
            v = plsc.load_gather(gtab_v, [gbase + (c * TL)])
            out_v[pl.ds(t * OUT_W + c * TL + r, L)] = v
        for c in range(AD):
            v = plsc.load_gather(atab_v, [abase + (c * TL)])
            out_v[pl.ds(t * OUT_W + (GD + c) * TL + r, L)] = v

    pltpu.sync_copy(out_v, out_hbm.at[pl.ds(wid * (TPW * OUT_W), TPW * OUT_W)])


_ctx_embed = functools.partial(
    pl.kernel,
    mesh=plsc.VectorSubcoreMesh(core_axis_name="c", subcore_axis_name="s"),
    out_type=jax.ShapeDtypeStruct((B // TL * OUT_W,), jnp.float32),
    scratch_types=[
        pltpu.VMEM((TPW * TOK_W,), jnp.int32),
        pltpu.VMEM((VP // TL * G_W,), jnp.float32),
        pltpu.VMEM((VP // TL * A_W,), jnp.float32),
        pltpu.VMEM((TPW * OUT_W,), jnp.float32),
    ],
    compiler_params=pltpu.CompilerParams(needs_layout_passes=False),
)(_body)


def _tiled_flat(x, rows, cols):
    return x.reshape(rows // TL, TL, cols).transpose(0, 2, 1).reshape(-1)


@jax.jit
def kernel(context_tokens, age_table, gender_table):
    tok = _tiled_flat(context_tokens.astype(jnp.int32), B, 2)
    gt = _tiled_flat(jnp.pad(gender_table, ((0, VP - VOCAB), (0, 0))), VP, GD)
    at = _tiled_flat(jnp.pad(age_table, ((0, VP - VOCAB), (0, 0))), VP, AD)
    flat = _ctx_embed(tok, gt, at)
    return (
        flat.reshape(B // TL, DP, TL)
        .transpose(0, 2, 1)
        .reshape(B, DP)[:, :D]
    )


# speedup vs baseline: 6.2099x; 1.0387x over previous
import functools

import jax
import jax.numpy as jnp
from jax import lax
from jax.experimental import pallas as pl
from jax.experimental.pallas import tpu as pltpu
from jax.experimental.pallas import tpu_sc as plsc

B = 16384
GD = 2
AD = 4
D = GD + AD
DP = 8
VOCAB = 1000
VP = 1024
TL = 128
L = 16

_info = plsc.get_sparse_core_info()
NC, NS = _info.num_cores, _info.num_subcores
NW = NC * NS
BPW = B // NW
TPW = BPW // TL
NGROUP = BPW // L

TOK_W = 2 * TL
OUT_W = DP * TL
G_W = GD * TL
A_W = AD * TL


def _body(tok_hbm, gtab_hbm, atab_hbm, out_hbm, tok_v, gtab_v, atab_v, out_v):
    wid = lax.axis_index("s") * NC + lax.axis_index("c")
    pltpu.sync_copy(tok_hbm.at[pl.ds(wid * (TPW * TOK_W), TPW * TOK_W)], tok_v)
    pltpu.sync_copy(gtab_hbm, gtab_v)
    pltpu.sync_copy(atab_hbm, atab_v)

    @pl.loop(0, NGROUP, unroll=4)
    def _group(g):
        t = g >> 3
        r = (g & 7) << 4
        toff = t * TOK_W + r
        ooff = t * OUT_W + r
        gidx = tok_v[pl.ds(toff, L)]
        aidx = tok_v[pl.ds(toff + TL, L)]
        gbase = (gidx >> 7) * G_W + (gidx & (TL - 1))
        abase = (aidx >> 7) * A_W + (aidx & (TL - 1))
        for c in range(GD):
            v = plsc.load_gather(gtab_v, [gbase + (c * TL)])
            out_v[pl.ds(ooff + c * TL, L)] = v
        for c in range(AD):
            v = plsc.load_gather(atab_v, [abase + (c * TL)])
            out_v[pl.ds(ooff + (GD + c) * TL, L)] = v

    pltpu.sync_copy(out_v, out_hbm.at[pl.ds(wid * (TPW * OUT_W), TPW * OUT_W)])


_ctx_embed = functools.partial(
    pl.kernel,
    mesh=plsc.VectorSubcoreMesh(core_axis_name="c", subcore_axis_name="s"),
    out_type=jax.ShapeDtypeStruct((B // TL * OUT_W,), jnp.float32),
    scratch_types=[
        pltpu.VMEM((TPW * TOK_W,), jnp.int32),
        pltpu.VMEM((VP // TL * G_W,), jnp.float32),
        pltpu.VMEM((VP // TL * A_W,), jnp.float32),
        pltpu.VMEM((TPW * OUT_W,), jnp.float32),
    ],
    compiler_params=pltpu.CompilerParams(needs_layout_passes=False),
)(_body)


def _tiled_flat(x, rows, cols):
    return x.reshape(rows // TL, TL, cols).transpose(0, 2, 1).reshape(-1)


@jax.jit
def kernel(context_tokens, age_table, gender_table):
    tok = _tiled_flat(context_tokens.astype(jnp.int32), B, 2)
    gt = _tiled_flat(jnp.pad(gender_table, ((0, VP - VOCAB), (0, 0))), VP, GD)
    at = _tiled_flat(jnp.pad(age_table, ((0, VP - VOCAB), (0, 0))), VP, AD)
    flat = _ctx_embed(tok, gt, at)
    return (
        flat.reshape(B // TL, DP, TL)
        .transpose(0, 2, 1)
        .reshape(B, DP)[:, :D]
    )


# speedup vs baseline: 6.4789x; 1.0433x over previous
import functools

import jax
import jax.numpy as jnp
from jax import lax
from jax.experimental import pallas as pl
from jax.experimental.pallas import tpu as pltpu
from jax.experimental.pallas import tpu_sc as plsc

B = 16384
GD = 2
AD = 4
D = GD + AD
DP = 8
VOCAB = 1000
VP = 1024
TL = 128
L = 16

_info = plsc.get_sparse_core_info()
NC, NS = _info.num_cores, _info.num_subcores
NW = NC * NS
BPW = B // NW
TPW = BPW // TL
NGROUP = BPW // L

TOK_W = 2 * TL
OUT_W = DP * TL
G_W = GD * TL
A_W = AD * TL


def _body(tok_hbm, gtab_hbm, atab_hbm, out_hbm, tok_v, gtab_v, atab_v, out_v,
          sem):
    wid = lax.axis_index("s") * NC + lax.axis_index("c")
    c1 = pltpu.async_copy(
        tok_hbm.at[pl.ds(wid * (TPW * TOK_W), TPW * TOK_W)], tok_v, sem)
    c2 = pltpu.async_copy(gtab_hbm, gtab_v, sem)
    c3 = pltpu.async_copy(atab_hbm, atab_v, sem)
    c1.wait()
    c2.wait()
    c3.wait()

    @pl.loop(0, NGROUP, unroll=4)
    def _group(g):
        t = g >> 3
        r = (g & 7) << 4
        toff = t * TOK_W + r
        ooff = t * OUT_W + r
        gidx = tok_v[pl.ds(toff, L)]
        aidx = tok_v[pl.ds(toff + TL, L)]
        gbase = (gidx >> 7) * G_W + (gidx & (TL - 1))
        abase = (aidx >> 7) * A_W + (aidx & (TL - 1))
        for c in range(GD):
            v = plsc.load_gather(gtab_v, [gbase + (c * TL)])
            out_v[pl.ds(ooff + c * TL, L)] = v
        for c in range(AD):
            v = plsc.load_gather(atab_v, [abase + (c * TL)])
            out_v[pl.ds(ooff + (GD + c) * TL, L)] = v

    pltpu.sync_copy(out_v, out_hbm.at[pl.ds(wid * (TPW * OUT_W), TPW * OUT_W)])


_ctx_embed = functools.partial(
    pl.kernel,
    mesh=plsc.VectorSubcoreMesh(core_axis_name="c", subcore_axis_name="s"),
    out_type=jax.ShapeDtypeStruct((B // TL * OUT_W,), jnp.float32),
    scratch_types=[
        pltpu.VMEM((TPW * TOK_W,), jnp.int32),
        pltpu.VMEM((VP // TL * G_W,), jnp.float32),
        pltpu.VMEM((VP // TL * A_W,), jnp.float32),
        pltpu.VMEM((TPW * OUT_W,), jnp.float32),
        pltpu.SemaphoreType.DMA,
    ],
    compiler_params=pltpu.CompilerParams(needs_layout_passes=False),
)(_body)


def _tiled_flat(x, rows, cols):
    return x.reshape(rows // TL, TL, cols).transpose(0, 2, 1).reshape(-1)


@jax.jit
def kernel(context_tokens, age_table, gender_table):
    tok = _tiled_flat(context_tokens.astype(jnp.int32), B, 2)
    gt = _tiled_flat(jnp.pad(gender_table, ((0, VP - VOCAB), (0, 0))), VP, GD)
    at = _tiled_flat(jnp.pad(age_table, ((0, VP - VOCAB), (0, 0))), VP, AD)
    flat = _ctx_embed(tok, gt, at)
    return (
        flat.reshape(B // TL, DP, TL)
        .transpose(0, 2, 1)
        .reshape(B, DP)[:, :D]
    )
